# baseline (speedup 1.0000x reference)
import functools

import jax
import jax.numpy as jnp
from jax import lax
from jax.experimental import pallas as pl
from jax.experimental.pallas import tpu as pltpu
from jax.experimental.pallas import tpu_sc as plsc

N = 10000
E = 320000
D = 128
NC = 2
NS = 16
NT = NC * NS

ET_DEG = E // NT

C = 125
NCH = (E // NT) // C
IB = 16
NB = NCH // IB
NPB = IB // 2
NP = 10240
RPT = NP // NS
ZR = 16

_SC_MESH = plsc.VectorSubcoreMesh(
    core_axis_name="c", subcore_axis_name="s", num_cores=NC, num_subcores=NS)


@functools.partial(
    pl.kernel,
    out_type=jax.ShapeDtypeStruct((NT * NP,), jnp.float32),
    mesh=_SC_MESH,
    scratch_types=[
        pltpu.VMEM((NCH, C), jnp.int32),
        pltpu.VMEM((N,), jnp.float32),
    ],
    compiler_params=pltpu.CompilerParams(needs_layout_passes=False),
)
def _sc_degrees(dst_hbm, out_hbm, dst_v, deg_v):
    c = lax.axis_index("c")
    s = lax.axis_index("s")
    wid = c * NS + s

    def zero_body(i, carry):
        deg_v[pl.ds(i * 16, 16)] = jnp.zeros((16,), jnp.float32)
        return carry

    lax.fori_loop(0, N // 16, zero_body, 0)

    pltpu.sync_copy(dst_hbm.at[wid], dst_v)

    ones = jnp.ones((16,), jnp.float32)
    tail = lax.iota(jnp.int32, 16) >= (16 - (C - (C // 16) * 16))

    def add_body(r, carry):
        for q in range(C // 16):
            idx = dst_v[r, pl.ds(q * 16, 16)]
            plsc.addupdate_scatter(deg_v, [idx], ones)
        idx = dst_v[r, pl.ds(C - 16, 16)]
        plsc.addupdate_scatter(deg_v, [idx], ones, mask=tail)
        return carry

    lax.fori_loop(0, NCH, add_body, 0)

    pltpu.sync_copy(deg_v, out_hbm.at[pl.ds(wid * NP, N)])


@functools.partial(
    pl.kernel,
    out_type=jax.ShapeDtypeStruct((NC, NP, D), jnp.float32),
    mesh=_SC_MESH,
    compiler_params=pltpu.CompilerParams(needs_layout_passes=False),
    scratch_types=[
        pltpu.VMEM((IB, C), jnp.int32),
        pltpu.VMEM((IB, C), jnp.int32),
        pltpu.VMEM((C, D), jnp.float32),
        pltpu.VMEM((C, D), jnp.float32),
        pltpu.VMEM((ZR, D), jnp.float32),
        pltpu.VMEM_SHARED((NP, D), jnp.float32),
        pltpu.SemaphoreType.DMA,
        pltpu.SemaphoreType.DMA,
        pltpu.SemaphoreType.DMA,
        pltpu.SemaphoreType.DMA,
    ],
)
def _sc_propagate(y_hbm, src_hbm, dst_hbm, out_hbm, src_v, dst_v,
                  rows_a, rows_b, z2, acc, sem_a, sem_b, ssem_a, ssem_b):
    c = lax.axis_index("c")
    s = lax.axis_index("s")
    w = c * NS + s

    pltpu.sync_copy(src_hbm.at[w, pl.ds(0, IB)], src_v)
    pltpu.sync_copy(dst_hbm.at[w, pl.ds(0, IB)], dst_v)
    pltpu.async_copy(y_hbm.at[src_v.at[0]], rows_a, sem_a)
    pltpu.async_copy(y_hbm.at[src_v.at[1]], rows_b, sem_b)

    for r in range(ZR):
        for q in range(D // 16):
            z2[r, pl.ds(q * 16, 16)] = jnp.zeros((16,), jnp.float32)
    rowbase = s * RPT

    def zcp(k, carry):
        pltpu.sync_copy(z2, acc.at[pl.ds(rowbase + k * ZR, ZR)])
        return carry

    lax.fori_loop(0, RPT // ZR, zcp, 0)
    plsc.subcore_barrier()

    for b in range(NB):
        if b > 0:
            pltpu.sync_copy(src_hbm.at[w, pl.ds(b * IB, IB)], src_v)
            pltpu.sync_copy(dst_hbm.at[w, pl.ds(b * IB, IB)], dst_v)
            pltpu.async_copy(y_hbm.at[src_v.at[0]], rows_a, sem_a)
            pltpu.async_copy(y_hbm.at[src_v.at[1]], rows_b, sem_b)

        def pair(k, carry):
            i0 = 2 * k
            i1 = i0 + 1
            pltpu.make_async_copy(y_hbm.at[src_v.at[i0]], rows_a, sem_a).wait()
            pltpu.async_copy(rows_a, acc.at[dst_v.at[i0]], ssem_a, add=True)

            @pl.when(k < NPB - 1)
            def _():
                pltpu.make_async_copy(rows_a, acc.at[dst_v.at[0]], ssem_a).wait()
                pltpu.async_copy(y_hbm.at[src_v.at[i0 + 2]], rows_a, sem_a)

            pltpu.make_async_copy(y_hbm.at[src_v.at[i1]], rows_b, sem_b).wait()
            pltpu.async_copy(rows_b, acc.at[dst_v.at[i1]], ssem_b, add=True)

            @pl.when(k < NPB - 1)
            def _():
                pltpu.make_async_copy(rows_b, acc.at[dst_v.at[0]], ssem_b).wait()
                pltpu.async_copy(y_hbm.at[src_v.at[i1 + 2]], rows_b, sem_b)
            return carry

        lax.fori_loop(0, NPB, pair, 0)
        pltpu.make_async_copy(rows_a, acc.at[dst_v.at[0]], ssem_a).wait()
        pltpu.make_async_copy(rows_b, acc.at[dst_v.at[0]], ssem_b).wait()
    plsc.subcore_barrier()

    pltpu.sync_copy(acc.at[pl.ds(rowbase, RPT)],
                    out_hbm.at[c, pl.ds(rowbase, RPT)])


RB = 2000


def _tc_prep_body(degp_ref, feats_ref, norm_ref, rnorm_ref, y_ref):
    degp = degp_ref[...].reshape(NT, NP)
    deg = jnp.sum(degp[:, :N], axis=0)
    deg = jnp.maximum(deg, 1.0)
    nrm = lax.rsqrt(deg)[:, None]
    norm_ref[...] = nrm
    rnorm_ref[...] = jnp.sqrt(deg)[:, None]
    y_ref[...] = feats_ref[...] * nrm


def _tc_hop_body(parts_ref, norm_ref, y_ref):
    x = parts_ref[...]
    p = x[0, :N] + x[1, :N]
    nrm = norm_ref[...]
    y_ref[...] = p * (nrm * nrm)


def _tc_final_body(y2_ref, y3_ref, parts3_ref, norm_ref, rnorm_ref,
                   W0_ref, b0_ref, W1_ref, b1_ref,
                   W2_ref, b2_ref, fcW_ref, fcb_ref, g_ref, bt_ref, out_ref):
    rn = rnorm_ref[...]
    h1 = (y2_ref[...] * rn).astype(jnp.bfloat16)
    h2 = (y3_ref[...] * rn).astype(jnp.bfloat16)
    x3 = parts3_ref[...]
    h3 = ((x3[0, :N] + x3[1, :N]) * norm_ref[...]).astype(jnp.bfloat16)
    W0b = W0_ref[...].astype(jnp.bfloat16)
    W1b = W1_ref[...].astype(jnp.bfloat16)
    W2b = W2_ref[...].astype(jnp.bfloat16)
    fcW = fcW_ref[...]
    M0 = jnp.dot(W0b, fcW[:D].astype(jnp.bfloat16),
                 preferred_element_type=jnp.float32).astype(jnp.bfloat16)
    M1 = jnp.dot(W1b, fcW[D:2 * D].astype(jnp.bfloat16),
                 preferred_element_type=jnp.float32).astype(jnp.bfloat16)
    M2 = jnp.dot(W2b, fcW[2 * D:].astype(jnp.bfloat16),
                 preferred_element_type=jnp.float32).astype(jnp.bfloat16)
    f = (jnp.dot(h1, M0, preferred_element_type=jnp.float32)
         + jnp.dot(h2, M1, preferred_element_type=jnp.float32)
         + jnp.dot(h3, M2, preferred_element_type=jnp.float32))
    fbias = (jnp.dot(b0_ref[...], fcW[:D], preferred_element_type=jnp.float32)
             + jnp.dot(b1_ref[...], fcW[D:2 * D], preferred_element_type=jnp.float32)
             + jnp.dot(b2_ref[...], fcW[2 * D:], preferred_element_type=jnp.float32)
             + fcb_ref[...])
    f = f + fbias
    mu = jnp.mean(f, axis=0, keepdims=True)
    var = jnp.mean((f - mu) ** 2, axis=0, keepdims=True)
    out_ref[...] = (f - mu) * lax.rsqrt(var + 1e-5) * g_ref[...] + bt_ref[...]


_tc_prep = pl.pallas_call(
    _tc_prep_body,
    out_shape=(jax.ShapeDtypeStruct((N, 1), jnp.float32),
               jax.ShapeDtypeStruct((N, 1), jnp.float32),
               jax.ShapeDtypeStruct((N, D), jnp.float32)),
)

_tc_hop = pl.pallas_call(
    _tc_hop_body,
    out_shape=jax.ShapeDtypeStruct((N, D), jnp.float32),
)

_tc_final = pl.pallas_call(
    _tc_final_body,
    out_shape=jax.ShapeDtypeStruct((N, D), jnp.float32),
)


def kernel(feats, edge_index, W0, b0, W1, b1, W2, b2, fc_W, fc_b,
           bn_gamma, bn_beta):
    src4 = edge_index[0].astype(jnp.int32).reshape(NT, NCH, C)
    dst4 = edge_index[1].astype(jnp.int32).reshape(NT, NCH, C)

    deg_flat = _sc_degrees(dst4)
    norm, rnorm, y1 = _tc_prep(deg_flat, feats)

    parts1 = _sc_propagate(y1, src4, dst4)
    y2 = _tc_hop(parts1, norm)
    parts2 = _sc_propagate(y2, src4, dst4)
    y3 = _tc_hop(parts2, norm)
    parts3 = _sc_propagate(y3, src4, dst4)

    return _tc_final(y2, y3, parts3, norm, rnorm,
                     W0, b0.reshape(1, D), W1, b1.reshape(1, D),
                     W2, b2.reshape(1, D), fc_W, fc_b.reshape(1, D),
                     bn_gamma.reshape(1, D), bn_beta.reshape(1, D))


# speedup vs baseline: 1.0138x; 1.0138x over previous
import functools

import jax
import jax.numpy as jnp
from jax import lax
from jax.experimental import pallas as pl
from jax.experimental.pallas import tpu as pltpu
from jax.experimental.pallas import tpu_sc as plsc

N = 10000
E = 320000
D = 128
NC = 2
NS = 16
NT = NC * NS

ET_DEG = E // NT

C = 125
NCH = (E // NT) // C
IB = 16
NB = NCH // IB
NPB = IB // 2
NP = 10240
RPT = NP // NS
ZR = 16

_SC_MESH = plsc.VectorSubcoreMesh(
    core_axis_name="c", subcore_axis_name="s", num_cores=NC, num_subcores=NS)


@functools.partial(
    pl.kernel,
    out_type=jax.ShapeDtypeStruct((NT * NP,), jnp.float32),
    mesh=_SC_MESH,
    scratch_types=[
        pltpu.VMEM((NCH, C), jnp.int32),
        pltpu.VMEM((N,), jnp.float32),
    ],
    compiler_params=pltpu.CompilerParams(needs_layout_passes=False),
)
def _sc_degrees(ei_hbm, out_hbm, dst_v, deg_v):
    c = lax.axis_index("c")
    s = lax.axis_index("s")
    wid = c * NS + s

    def zero_body(i, carry):
        deg_v[pl.ds(i * 16, 16)] = jnp.zeros((16,), jnp.float32)
        return carry

    lax.fori_loop(0, N // 16, zero_body, 0)

    pltpu.sync_copy(ei_hbm.at[1, wid], dst_v)

    ones = jnp.ones((16,), jnp.float32)
    tail = lax.iota(jnp.int32, 16) >= (16 - (C - (C // 16) * 16))

    def add_body(r, carry):
        for q in range(C // 16):
            idx = dst_v[r, pl.ds(q * 16, 16)]
            plsc.addupdate_scatter(deg_v, [idx], ones)
        idx = dst_v[r, pl.ds(C - 16, 16)]
        plsc.addupdate_scatter(deg_v, [idx], ones, mask=tail)
        return carry

    lax.fori_loop(0, NCH, add_body, 0)

    pltpu.sync_copy(deg_v, out_hbm.at[pl.ds(wid * NP, N)])


@functools.partial(
    pl.kernel,
    out_type=jax.ShapeDtypeStruct((NC * NP, D), jnp.float32),
    mesh=_SC_MESH,
    compiler_params=pltpu.CompilerParams(needs_layout_passes=False),
    scratch_types=[
        pltpu.VMEM((IB, C), jnp.int32),
        pltpu.VMEM((IB, C), jnp.int32),
        pltpu.VMEM((C, D), jnp.float32),
        pltpu.VMEM((C, D), jnp.float32),
        pltpu.VMEM((ZR, D), jnp.float32),
        pltpu.VMEM_SHARED((NP, D), jnp.float32),
        pltpu.SemaphoreType.DMA,
        pltpu.SemaphoreType.DMA,
        pltpu.SemaphoreType.DMA,
        pltpu.SemaphoreType.DMA,
    ],
)
def _sc_propagate(y_hbm, ei_hbm, out_hbm, src_v, dst_v,
                  rows_a, rows_b, z2, acc, sem_a, sem_b, ssem_a, ssem_b):
    c = lax.axis_index("c")
    s = lax.axis_index("s")
    w = c * NS + s

    pltpu.sync_copy(ei_hbm.at[0, w, pl.ds(0, IB)], src_v)
    pltpu.sync_copy(ei_hbm.at[1, w, pl.ds(0, IB)], dst_v)
    pltpu.async_copy(y_hbm.at[src_v.at[0]], rows_a, sem_a)
    pltpu.async_copy(y_hbm.at[src_v.at[1]], rows_b, sem_b)

    for r in range(ZR):
        for q in range(D // 16):
            z2[r, pl.ds(q * 16, 16)] = jnp.zeros((16,), jnp.float32)
    rowbase = s * RPT

    def zcp(k, carry):
        pltpu.sync_copy(z2, acc.at[pl.ds(rowbase + k * ZR, ZR)])
        return carry

    lax.fori_loop(0, RPT // ZR, zcp, 0)
    plsc.subcore_barrier()

    for b in range(NB):
        if b > 0:
            pltpu.sync_copy(ei_hbm.at[0, w, pl.ds(b * IB, IB)], src_v)
            pltpu.sync_copy(ei_hbm.at[1, w, pl.ds(b * IB, IB)], dst_v)
            pltpu.async_copy(y_hbm.at[src_v.at[0]], rows_a, sem_a)
            pltpu.async_copy(y_hbm.at[src_v.at[1]], rows_b, sem_b)

        def pair(k, carry):
            i0 = 2 * k
            i1 = i0 + 1
            pltpu.make_async_copy(y_hbm.at[src_v.at[i0]], rows_a, sem_a).wait()
            pltpu.async_copy(rows_a, acc.at[dst_v.at[i0]], ssem_a, add=True)

            @pl.when(k < NPB - 1)
            def _():
                pltpu.make_async_copy(rows_a, acc.at[dst_v.at[0]], ssem_a).wait()
                pltpu.async_copy(y_hbm.at[src_v.at[i0 + 2]], rows_a, sem_a)

            pltpu.make_async_copy(y_hbm.at[src_v.at[i1]], rows_b, sem_b).wait()
            pltpu.async_copy(rows_b, acc.at[dst_v.at[i1]], ssem_b, add=True)

            @pl.when(k < NPB - 1)
            def _():
                pltpu.make_async_copy(rows_b, acc.at[dst_v.at[0]], ssem_b).wait()
                pltpu.async_copy(y_hbm.at[src_v.at[i1 + 2]], rows_b, sem_b)
            return carry

        lax.fori_loop(0, NPB, pair, 0)
        pltpu.make_async_copy(rows_a, acc.at[dst_v.at[0]], ssem_a).wait()
        pltpu.make_async_copy(rows_b, acc.at[dst_v.at[0]], ssem_b).wait()
    plsc.subcore_barrier()

    pltpu.sync_copy(acc.at[pl.ds(rowbase, RPT)],
                    out_hbm.at[pl.ds(c * NP + rowbase, RPT)])


RB = 2000


def _tc_prep_body(degp_ref, feats_ref, norm_ref, rnorm_ref, y_ref):
    deg = jnp.sum(degp_ref[...][:, :N], axis=0)
    deg = jnp.maximum(deg, 1.0)
    nrm = lax.rsqrt(deg)[:, None]
    norm_ref[...] = nrm
    rnorm_ref[...] = jnp.sqrt(deg)[:, None]
    y_ref[...] = feats_ref[...] * nrm


def _tc_hop_body(parts_ref, norm_ref, y_ref):
    x = parts_ref[...]
    p = x[:N] + x[NP:NP + N]
    nrm = norm_ref[...]
    y_ref[...] = p * (nrm * nrm)


def _tc_final_body(y2_ref, y3_ref, parts3_ref, norm_ref, rnorm_ref,
                   W0_ref, b0_ref, W1_ref, b1_ref,
                   W2_ref, b2_ref, fcW_ref, fcb_ref, g_ref, bt_ref, out_ref):
    rn = rnorm_ref[...]
    h1 = (y2_ref[...] * rn).astype(jnp.bfloat16)
    h2 = (y3_ref[...] * rn).astype(jnp.bfloat16)
    x3 = parts3_ref[...]
    h3 = ((x3[:N] + x3[NP:NP + N]) * norm_ref[...]).astype(jnp.bfloat16)
    W0b = W0_ref[...].astype(jnp.bfloat16)
    W1b = W1_ref[...].astype(jnp.bfloat16)
    W2b = W2_ref[...].astype(jnp.bfloat16)
    fcW = fcW_ref[...]
    M0 = jnp.dot(W0b, fcW[:D].astype(jnp.bfloat16),
                 preferred_element_type=jnp.float32).astype(jnp.bfloat16)
    M1 = jnp.dot(W1b, fcW[D:2 * D].astype(jnp.bfloat16),
                 preferred_element_type=jnp.float32).astype(jnp.bfloat16)
    M2 = jnp.dot(W2b, fcW[2 * D:].astype(jnp.bfloat16),
                 preferred_element_type=jnp.float32).astype(jnp.bfloat16)
    f = (jnp.dot(h1, M0, preferred_element_type=jnp.float32)
         + jnp.dot(h2, M1, preferred_element_type=jnp.float32)
         + jnp.dot(h3, M2, preferred_element_type=jnp.float32))
    fbias = (jnp.dot(b0_ref[...], fcW[:D], preferred_element_type=jnp.float32)
             + jnp.dot(b1_ref[...], fcW[D:2 * D], preferred_element_type=jnp.float32)
             + jnp.dot(b2_ref[...], fcW[2 * D:], preferred_element_type=jnp.float32)
             + fcb_ref[...])
    f = f + fbias
    mu = jnp.mean(f, axis=0, keepdims=True)
    var = jnp.mean((f - mu) ** 2, axis=0, keepdims=True)
    out_ref[...] = (f - mu) * lax.rsqrt(var + 1e-5) * g_ref[...] + bt_ref[...]


_tc_prep = pl.pallas_call(
    _tc_prep_body,
    out_shape=(jax.ShapeDtypeStruct((N, 1), jnp.float32),
               jax.ShapeDtypeStruct((N, 1), jnp.float32),
               jax.ShapeDtypeStruct((N, D), jnp.float32)),
)

_tc_hop = pl.pallas_call(
    _tc_hop_body,
    out_shape=jax.ShapeDtypeStruct((N, D), jnp.float32),
)

_tc_final = pl.pallas_call(
    _tc_final_body,
    out_shape=jax.ShapeDtypeStruct((N, D), jnp.float32),
)


def kernel(feats, edge_index, W0, b0, W1, b1, W2, b2, fc_W, fc_b,
           bn_gamma, bn_beta):
    ei = edge_index.astype(jnp.int32).reshape(2, NT, NCH, C)

    deg_parts = _sc_degrees(ei).reshape(NT, NP)
    norm, rnorm, y1 = _tc_prep(deg_parts, feats)

    parts1 = _sc_propagate(y1, ei)
    y2 = _tc_hop(parts1, norm)
    parts2 = _sc_propagate(y2, ei)
    y3 = _tc_hop(parts2, norm)
    parts3 = _sc_propagate(y3, ei)

    return _tc_final(y2, y3, parts3, norm, rnorm,
                     W0, b0.reshape(1, D), W1, b1.reshape(1, D),
                     W2, b2.reshape(1, D), fc_W, fc_b.reshape(1, D),
                     bn_gamma.reshape(1, D), bn_beta.reshape(1, D))


# speedup vs baseline: 1.0159x; 1.0020x over previous
import functools

import jax
import jax.numpy as jnp
from jax import lax
from jax.experimental import pallas as pl
from jax.experimental.pallas import tpu as pltpu
from jax.experimental.pallas import tpu_sc as plsc

N = 10000
E = 320000
D = 128
NC = 2
NS = 16
NT = NC * NS

ET_DEG = E // NT

C = 125
NCH = (E // NT) // C
IB = 16
NB = NCH // IB
NPB = IB // 2
NP = 10240
RPT = NP // NS
ZR = 16

_SC_MESH = plsc.VectorSubcoreMesh(
    core_axis_name="c", subcore_axis_name="s", num_cores=NC, num_subcores=NS)


@functools.partial(
    pl.kernel,
    out_type=jax.ShapeDtypeStruct((NT * NP,), jnp.float32),
    mesh=_SC_MESH,
    scratch_types=[
        pltpu.VMEM((NCH, C), jnp.int32),
        pltpu.VMEM((N,), jnp.float32),
    ],
    compiler_params=pltpu.CompilerParams(needs_layout_passes=False),
)
def _sc_degrees(ei_hbm, out_hbm, dst_v, deg_v):
    c = lax.axis_index("c")
    s = lax.axis_index("s")
    wid = c * NS + s

    def zero_body(i, carry):
        deg_v[pl.ds(i * 16, 16)] = jnp.zeros((16,), jnp.float32)
        return carry

    lax.fori_loop(0, N // 16, zero_body, 0)

    pltpu.sync_copy(ei_hbm.at[1, wid], dst_v)

    ones = jnp.ones((16,), jnp.float32)
    tail = lax.iota(jnp.int32, 16) >= (16 - (C - (C // 16) * 16))

    def add_body(r, carry):
        for q in range(C // 16):
            idx = dst_v[r, pl.ds(q * 16, 16)]
            plsc.addupdate_scatter(deg_v, [idx], ones)
        idx = dst_v[r, pl.ds(C - 16, 16)]
        plsc.addupdate_scatter(deg_v, [idx], ones, mask=tail)
        return carry

    lax.fori_loop(0, NCH, add_body, 0)

    pltpu.sync_copy(deg_v, out_hbm.at[pl.ds(wid * NP, N)])


@functools.partial(
    pl.kernel,
    out_type=jax.ShapeDtypeStruct((NC * NP, D), jnp.float32),
    mesh=_SC_MESH,
    compiler_params=pltpu.CompilerParams(needs_layout_passes=False),
    scratch_types=[
        pltpu.VMEM((IB, C), jnp.int32),
        pltpu.VMEM((IB, C), jnp.int32),
        pltpu.VMEM((C, D), jnp.float32),
        pltpu.VMEM((C, D), jnp.float32),
        pltpu.VMEM((ZR, D), jnp.float32),
        pltpu.VMEM_SHARED((NP, D), jnp.float32),
        pltpu.SemaphoreType.DMA,
        pltpu.SemaphoreType.DMA,
        pltpu.SemaphoreType.DMA,
        pltpu.SemaphoreType.DMA,
    ],
)
def _sc_propagate(y_hbm, ei_hbm, out_hbm, src_v, dst_v,
                  rows_a, rows_b, z2, acc, sem_a, sem_b, ssem_a, ssem_b):
    c = lax.axis_index("c")
    s = lax.axis_index("s")
    w = c * NS + s

    pltpu.sync_copy(ei_hbm.at[0, w, pl.ds(0, IB)], src_v)
    pltpu.sync_copy(ei_hbm.at[1, w, pl.ds(0, IB)], dst_v)
    pltpu.async_copy(y_hbm.at[src_v.at[0]], rows_a, sem_a)
    pltpu.async_copy(y_hbm.at[src_v.at[1]], rows_b, sem_b)

    for r in range(ZR):
        for q in range(D // 16):
            z2[r, pl.ds(q * 16, 16)] = jnp.zeros((16,), jnp.float32)
    rowbase = s * RPT

    def zcp(k, carry):
        pltpu.sync_copy(z2, acc.at[pl.ds(rowbase + k * ZR, ZR)])
        return carry

    lax.fori_loop(0, RPT // ZR, zcp, 0)
    plsc.subcore_barrier()

    for b in range(NB):
        if b > 0:
            pltpu.sync_copy(ei_hbm.at[0, w, pl.ds(b * IB, IB)], src_v)
            pltpu.sync_copy(ei_hbm.at[1, w, pl.ds(b * IB, IB)], dst_v)
            pltpu.async_copy(y_hbm.at[src_v.at[0]], rows_a, sem_a)
            pltpu.async_copy(y_hbm.at[src_v.at[1]], rows_b, sem_b)

        def pair(k, carry):
            i0 = 2 * k
            i1 = i0 + 1
            pltpu.make_async_copy(y_hbm.at[src_v.at[i0]], rows_a, sem_a).wait()
            pltpu.async_copy(rows_a, acc.at[dst_v.at[i0]], ssem_a, add=True)

            @pl.when(k < NPB - 1)
            def _():
                pltpu.make_async_copy(rows_a, acc.at[dst_v.at[0]], ssem_a).wait()
                pltpu.async_copy(y_hbm.at[src_v.at[i0 + 2]], rows_a, sem_a)

            pltpu.make_async_copy(y_hbm.at[src_v.at[i1]], rows_b, sem_b).wait()
            pltpu.async_copy(rows_b, acc.at[dst_v.at[i1]], ssem_b, add=True)

            @pl.when(k < NPB - 1)
            def _():
                pltpu.make_async_copy(rows_b, acc.at[dst_v.at[0]], ssem_b).wait()
                pltpu.async_copy(y_hbm.at[src_v.at[i1 + 2]], rows_b, sem_b)
            return carry

        lax.fori_loop(0, NPB, pair, 0)
        pltpu.make_async_copy(rows_a, acc.at[dst_v.at[0]], ssem_a).wait()
        pltpu.make_async_copy(rows_b, acc.at[dst_v.at[0]], ssem_b).wait()
    plsc.subcore_barrier()

    pltpu.sync_copy(acc.at[pl.ds(rowbase, RPT)],
                    out_hbm.at[pl.ds(c * NP + rowbase, RPT)])


def _tc_prep_body(degp_ref, feats_ref, norm_ref, rnorm_ref, y_ref):
    deg = jnp.sum(degp_ref[...][:, :N], axis=0)
    deg = jnp.maximum(deg, 1.0)
    nrm = lax.rsqrt(deg)[:, None]
    norm_ref[...] = nrm
    rnorm_ref[...] = jnp.sqrt(deg)[:, None]
    y_ref[...] = feats_ref[...] * nrm


def _tc_hop_body(parts_ref, norm_ref, y_ref):
    x = parts_ref[...]
    p = x[:N] + x[NP:NP + N]
    nrm = norm_ref[...]
    y_ref[...] = p * (nrm * nrm)


def _tc_final_body(y2_ref, y3_ref, parts3_ref, norm_ref, rnorm_ref,
                   W0_ref, b0_ref, W1_ref, b1_ref,
                   W2_ref, b2_ref, fcW_ref, fcb_ref, g_ref, bt_ref, out_ref):
    rn = rnorm_ref[...]
    h1 = (y2_ref[...] * rn).astype(jnp.bfloat16)
    h2 = (y3_ref[...] * rn).astype(jnp.bfloat16)
    x3 = parts3_ref[...]
    h3 = ((x3[:N] + x3[NP:NP + N]) * norm_ref[...]).astype(jnp.bfloat16)
    W0b = W0_ref[...].astype(jnp.bfloat16)
    W1b = W1_ref[...].astype(jnp.bfloat16)
    W2b = W2_ref[...].astype(jnp.bfloat16)
    fcW = fcW_ref[...]
    M0 = jnp.dot(W0b, fcW[:D].astype(jnp.bfloat16),
                 preferred_element_type=jnp.float32).astype(jnp.bfloat16)
    M1 = jnp.dot(W1b, fcW[D:2 * D].astype(jnp.bfloat16),
                 preferred_element_type=jnp.float32).astype(jnp.bfloat16)
    M2 = jnp.dot(W2b, fcW[2 * D:].astype(jnp.bfloat16),
                 preferred_element_type=jnp.float32).astype(jnp.bfloat16)
    f = (jnp.dot(h1, M0, preferred_element_type=jnp.float32)
         + jnp.dot(h2, M1, preferred_element_type=jnp.float32)
         + jnp.dot(h3, M2, preferred_element_type=jnp.float32))
    fbias = (jnp.dot(b0_ref[...], fcW[:D], preferred_element_type=jnp.float32)
             + jnp.dot(b1_ref[...], fcW[D:2 * D], preferred_element_type=jnp.float32)
             + jnp.dot(b2_ref[...], fcW[2 * D:], preferred_element_type=jnp.float32)
             + fcb_ref[...])
    f = f + fbias
    mu = jnp.mean(f, axis=0, keepdims=True)
    var = jnp.mean((f - mu) ** 2, axis=0, keepdims=True)
    out_ref[...] = (f - mu) * lax.rsqrt(var + 1e-5) * g_ref[...] + bt_ref[...]


_tc_prep = pl.pallas_call(
    _tc_prep_body,
    out_shape=(jax.ShapeDtypeStruct((N, 1), jnp.float32),
               jax.ShapeDtypeStruct((N, 1), jnp.float32),
               jax.ShapeDtypeStruct((N, D), jnp.float32)),
)

_tc_hop = pl.pallas_call(
    _tc_hop_body,
    out_shape=jax.ShapeDtypeStruct((N, D), jnp.float32),
)

_tc_final = pl.pallas_call(
    _tc_final_body,
    out_shape=jax.ShapeDtypeStruct((N, D), jnp.float32),
)


def kernel(feats, edge_index, W0, b0, W1, b1, W2, b2, fc_W, fc_b,
           bn_gamma, bn_beta):
    ei = edge_index.astype(jnp.int32).reshape(2, NT, NCH, C)

    deg_parts = _sc_degrees(ei).reshape(NT, NP)
    norm, rnorm, y1 = _tc_prep(deg_parts, feats)

    parts1 = _sc_propagate(y1, ei)
    y2 = _tc_hop(parts1, norm)
    parts2 = _sc_propagate(y2, ei)
    y3 = _tc_hop(parts2, norm)
    parts3 = _sc_propagate(y3, ei)

    return _tc_final(y2, y3, parts3, norm, rnorm,
                     W0, b0.reshape(1, D), W1, b1.reshape(1, D),
                     W2, b2.reshape(1, D), fc_W, fc_b.reshape(1, D),
                     bn_gamma.reshape(1, D), bn_beta.reshape(1, D))
